# baseline (speedup 1.0000x reference)
import functools

import jax
import jax.numpy as jnp
from jax import lax
from jax.experimental import pallas as pl
from jax.experimental.pallas import tpu as pltpu
from jax.experimental.pallas import tpu_sc as plsc

_NUM_CORES = 2
_NUM_SUBCORES = 16
_NUM_WORKERS = _NUM_CORES * _NUM_SUBCORES
_LANES = 16

_N = 200000
_C = 64
_B = 4
_H = 256
_W = 256
_NZERO = 1024
_ZERO_ROW = _B * _H * _W

_K = 800
_NCHUNKS = _N // _K
_CHUNKS_PER_WORKER = -(-_NCHUNKS // _NUM_WORKERS)
_GSUB = 80
_NGATHER = _K // _GSUB
_VPERROW = _C // _LANES


def _make_sc_kernel():
    mesh = plsc.VectorSubcoreMesh(core_axis_name="c", subcore_axis_name="s")

    @functools.partial(
        pl.kernel,
        out_type=jax.ShapeDtypeStruct((_N, _C), jnp.float32),
        mesh=mesh,
        compiler_params=pltpu.CompilerParams(use_tc_tiling_on_sc=False),
        scratch_types=[
            pltpu.VMEM((_K, _C), jnp.float32),
            pltpu.VMEM((_K, _C), jnp.float32),
            pltpu.VMEM((_K,), jnp.int32),
            pltpu.VMEM((_K,), jnp.int32),
            pltpu.VMEM((_K,), jnp.int32),
            pltpu.VMEM((_K,), jnp.int32),
            pltpu.VMEM((_NGATHER, _GSUB), jnp.int32),
            pltpu.SemaphoreType.DMA,
            pltpu.SemaphoreType.DMA,
        ],
    )
    def sc_kernel(vf_hbm, table_hbm, gx_hbm, gy_hbm, b_hbm, m_hbm, out_hbm,
                  a_v, g_v, gx_v, gy_v, b_v, m_v, r_v, vf_sem, g_sem):
        wid = lax.axis_index("s") * _NUM_CORES + lax.axis_index("c")

        for t in range(_CHUNKS_PER_WORKER):
            cid = wid + t * _NUM_WORKERS

            @pl.when(cid < _NCHUNKS)
            def _chunk():
                base = cid * _K
                rows = pl.ds(base, _K)

                vf_copy = pltpu.async_copy(vf_hbm.at[rows, :], a_v, vf_sem)

                pltpu.sync_copy(gx_hbm.at[rows], gx_v)
                pltpu.sync_copy(gy_hbm.at[rows], gy_v)
                pltpu.sync_copy(b_hbm.at[rows], b_v)
                pltpu.sync_copy(m_hbm.at[rows], m_v)

                def rbody(j, carry):
                    sl = pl.ds(j * _LANES, _LANES)
                    x16 = gx_v[sl]
                    y16 = gy_v[sl]
                    b16 = b_v[sl]
                    m16 = m_v[sl]
                    r16 = b16 * (_H * _W) + y16 * _W + x16
                    zero16 = (_ZERO_ROW
                              + ((j * _LANES + lax.iota(jnp.int32, _LANES))
                                 & (_NZERO - 1)))
                    r16 = jnp.where(m16 != 0, r16, zero16)
                    nvr = _GSUB // _LANES
                    r_v[j // nvr, pl.ds((j % nvr) * _LANES, _LANES)] = r16
                    return carry

                lax.fori_loop(0, _K // _LANES, rbody, 0)

                vf_copy.wait()
                handles = []
                for j in range(_NGATHER):
                    handles.append(
                        pltpu.async_copy(
                            table_hbm.at[r_v.at[j]],
                            a_v.at[pl.ds(j * _GSUB, _GSUB), :],
                            g_sem,
                            add=True,
                        ))
                for h in handles:
                    h.wait()

                pltpu.sync_copy(a_v, out_hbm.at[rows, :])

    return sc_kernel


_sc_kernel = _make_sc_kernel()


def kernel(voxel_features, image_feat, image_grid, batch_idx, point_mask):
    table = jnp.transpose(image_feat, (0, 2, 3, 1)).reshape(_B * _H * _W, _C)
    table = jnp.pad(table, ((0, _NZERO), (0, 0)))
    gx = image_grid[:, 0].astype(jnp.int32)
    gy = image_grid[:, 1].astype(jnp.int32)
    bi = batch_idx.astype(jnp.int32)
    m = point_mask.astype(jnp.int32)
    return _sc_kernel(voxel_features, table, gx, gy, bi, m)


# speedup vs baseline: 1.2285x; 1.2285x over previous
import functools

import jax
import jax.numpy as jnp
from jax import lax
from jax.experimental import pallas as pl
from jax.experimental.pallas import tpu as pltpu
from jax.experimental.pallas import tpu_sc as plsc

_NUM_CORES = 2
_NUM_SUBCORES = 16
_NUM_WORKERS = _NUM_CORES * _NUM_SUBCORES
_LANES = 16

_N = 200000
_C = 64
_B = 4
_H = 256
_W = 256
_CT = 128
_NZERO = 1024
_ZERO_ROW = _B * _H * _W

_K = 400
_NCHUNKS = _N // _K
_CHUNKS_PER_WORKER = -(-_NCHUNKS // _NUM_WORKERS)
_GSUB = 80
_NGATHER = _K // _GSUB
_VPERROW = _C // _LANES


def _make_sc_kernel():
    mesh = plsc.VectorSubcoreMesh(core_axis_name="c", subcore_axis_name="s")

    @functools.partial(
        pl.kernel,
        out_type=jax.ShapeDtypeStruct((_N, _C), jnp.float32),
        mesh=mesh,
        scratch_types=[
            pltpu.VMEM((_K, _C), jnp.float32),
            pltpu.VMEM((_K, _CT), jnp.float32),
            pltpu.VMEM((_K,), jnp.int32),
            pltpu.VMEM((_K,), jnp.int32),
            pltpu.VMEM((_K,), jnp.int32),
            pltpu.VMEM((_K,), jnp.int32),
            pltpu.VMEM((_NGATHER, _GSUB), jnp.int32),
            pltpu.SemaphoreType.DMA,
            pltpu.SemaphoreType.DMA,
        ],
    )
    def sc_kernel(vf_hbm, table_hbm, gx_hbm, gy_hbm, b_hbm, m_hbm, out_hbm,
                  v_v, g_v, gx_v, gy_v, b_v, m_v, r_v, vf_sem, g_sem):
        wid = lax.axis_index("s") * _NUM_CORES + lax.axis_index("c")

        for t in range(_CHUNKS_PER_WORKER):
            cid = wid + t * _NUM_WORKERS

            @pl.when(cid < _NCHUNKS)
            def _chunk():
                base = cid * _K
                rows = pl.ds(base, _K)

                vf_copy = pltpu.async_copy(vf_hbm.at[rows, :], v_v, vf_sem)

                pltpu.sync_copy(gx_hbm.at[rows], gx_v)
                pltpu.sync_copy(gy_hbm.at[rows], gy_v)
                pltpu.sync_copy(b_hbm.at[rows], b_v)
                pltpu.sync_copy(m_hbm.at[rows], m_v)

                def rbody(j, carry):
                    sl = pl.ds(j * _LANES, _LANES)
                    x16 = gx_v[sl]
                    y16 = gy_v[sl]
                    b16 = b_v[sl]
                    m16 = m_v[sl]
                    r16 = b16 * (_H * _W) + y16 * _W + x16
                    zero16 = (_ZERO_ROW
                              + ((j * _LANES + lax.iota(jnp.int32, _LANES))
                                 & (_NZERO - 1)))
                    r16 = jnp.where(m16 != 0, r16, zero16)
                    nvr = _GSUB // _LANES
                    r_v[j // nvr, pl.ds((j % nvr) * _LANES, _LANES)] = r16
                    return carry

                lax.fori_loop(0, _K // _LANES, rbody, 0)

                handles = []
                for j in range(_NGATHER):
                    handles.append(
                        pltpu.async_copy(
                            table_hbm.at[r_v.at[j]],
                            g_v.at[pl.ds(j * _GSUB, _GSUB), :],
                            g_sem,
                        ))
                vf_copy.wait()
                for h in handles:
                    h.wait()

                def fbody(i, carry):
                    for s in range(_VPERROW):
                        sl = pl.ds(s * _LANES, _LANES)
                        plsc.addupdate(v_v.at[i, sl], g_v[i, sl])
                    return carry

                lax.fori_loop(0, _K, fbody, 0)

                pltpu.sync_copy(v_v, out_hbm.at[rows, :])

    return sc_kernel


_sc_kernel = _make_sc_kernel()


def kernel(voxel_features, image_feat, image_grid, batch_idx, point_mask):
    feats = jnp.transpose(image_feat, (0, 2, 3, 1)).reshape(_B * _H * _W, _C)
    table = jnp.zeros((_B * _H * _W + _NZERO, _CT), jnp.float32)
    table = lax.dynamic_update_slice(table, feats, (0, 0))
    gx = image_grid[:, 0].astype(jnp.int32)
    gy = image_grid[:, 1].astype(jnp.int32)
    bi = batch_idx.astype(jnp.int32)
    m = point_mask.astype(jnp.int32)
    return _sc_kernel(voxel_features, table, gx, gy, bi, m)


# speedup vs baseline: 1.3139x; 1.0695x over previous
import functools

import jax
import jax.numpy as jnp
from jax import lax
from jax.experimental import pallas as pl
from jax.experimental.pallas import tpu as pltpu
from jax.experimental.pallas import tpu_sc as plsc

_NUM_CORES = 2
_NUM_SUBCORES = 16
_NUM_WORKERS = _NUM_CORES * _NUM_SUBCORES
_LANES = 16

_N = 200000
_C = 64
_B = 4
_H = 256
_W = 256
_CT = 128
_NZERO = 1024
_ZERO_ROW = _B * _H * _W

_K = 400
_NCHUNKS = _N // _K
_CHUNKS_PER_WORKER = -(-_NCHUNKS // _NUM_WORKERS)
_GSUB = 80
_NGATHER = _K // _GSUB
_VPERROW = _C // _LANES


def _make_sc_kernel():
    mesh = plsc.VectorSubcoreMesh(core_axis_name="c", subcore_axis_name="s")

    @functools.partial(
        pl.kernel,
        out_type=jax.ShapeDtypeStruct((_N, _C), jnp.float32),
        mesh=mesh,
        scratch_types=[
            pltpu.VMEM((_K, _C), jnp.float32),
            pltpu.VMEM((_K, _CT), jnp.float32),
            pltpu.VMEM((_K,), jnp.int32),
            pltpu.VMEM((_K,), jnp.int32),
            pltpu.VMEM((_K,), jnp.int32),
            pltpu.VMEM((_K,), jnp.int32),
            pltpu.VMEM((_NGATHER, _GSUB), jnp.int32),
            pltpu.SemaphoreType.DMA,
            pltpu.SemaphoreType.DMA,
            pltpu.SemaphoreType.DMA,
        ],
    )
    def sc_kernel(vf_hbm, table_hbm, gx_hbm, gy_hbm, b_hbm, m_hbm, out_hbm,
                  v_v, g_v, gx_v, gy_v, b_v, m_v, r_v, vf_sem, g_sem, wb_sem):
        wid = lax.axis_index("s") * _NUM_CORES + lax.axis_index("c")

        pending_wb = []

        for t in range(_CHUNKS_PER_WORKER):
            cid = wid + t * _NUM_WORKERS
            base = cid * _K
            rows = pl.ds(base, _K)
            vf_d = pltpu.make_async_copy(vf_hbm.at[rows, :], v_v, vf_sem)
            g_ds = [
                pltpu.make_async_copy(
                    table_hbm.at[r_v.at[j]],
                    g_v.at[pl.ds(j * _GSUB, _GSUB), :],
                    g_sem,
                ) for j in range(_NGATHER)
            ]
            wb_ds = [
                pltpu.make_async_copy(
                    v_v.at[pl.ds(j * _GSUB, _GSUB), :],
                    out_hbm.at[pl.ds(base + j * _GSUB, _GSUB), :],
                    wb_sem,
                ) for j in range(_NGATHER)
            ]

            @pl.when(cid < _NCHUNKS)
            def _chunk():
                for d in pending_wb:
                    d.wait()

                vf_d.start()

                pltpu.sync_copy(gx_hbm.at[rows], gx_v)
                pltpu.sync_copy(gy_hbm.at[rows], gy_v)
                pltpu.sync_copy(b_hbm.at[rows], b_v)
                pltpu.sync_copy(m_hbm.at[rows], m_v)

                @plsc.parallel_loop(0, _K // _LANES, unroll=2)
                def rbody(j):
                    sl = pl.ds(j * _LANES, _LANES)
                    x16 = gx_v[sl]
                    y16 = gy_v[sl]
                    b16 = b_v[sl]
                    m16 = m_v[sl]
                    r16 = b16 * (_H * _W) + y16 * _W + x16
                    zero16 = (_ZERO_ROW
                              + ((j * _LANES + lax.iota(jnp.int32, _LANES))
                                 & (_NZERO - 1)))
                    r16 = jnp.where(m16 != 0, r16, zero16)
                    nvr = _GSUB // _LANES
                    r_v[j // nvr, pl.ds((j % nvr) * _LANES, _LANES)] = r16

                for d in g_ds:
                    d.start()
                vf_d.wait()

                for j in range(_NGATHER):
                    g_ds[j].wait()

                    @plsc.parallel_loop(j * _GSUB, (j + 1) * _GSUB, unroll=4)
                    def fbody(i):
                        for s in range(_VPERROW):
                            sl = pl.ds(s * _LANES, _LANES)
                            plsc.addupdate(v_v.at[i, sl], g_v[i, sl])

                    wb_ds[j].start()

            pending_wb = wb_ds

        for d in pending_wb:
            d.wait()

    return sc_kernel


_sc_kernel = _make_sc_kernel()


def kernel(voxel_features, image_feat, image_grid, batch_idx, point_mask):
    feats = jnp.transpose(image_feat, (0, 2, 3, 1)).reshape(_B * _H * _W, _C)
    table = jnp.zeros((_B * _H * _W + _NZERO, _CT), jnp.float32)
    table = lax.dynamic_update_slice(table, feats, (0, 0))
    gx = image_grid[:, 0].astype(jnp.int32)
    gy = image_grid[:, 1].astype(jnp.int32)
    bi = batch_idx.astype(jnp.int32)
    m = point_mask.astype(jnp.int32)
    return _sc_kernel(voxel_features, table, gx, gy, bi, m)
